# baseline (speedup 1.0000x reference)
import functools

import jax
import jax.numpy as jnp
from jax import lax
from jax.experimental import pallas as pl
from jax.experimental.pallas import tpu as pltpu
from jax.experimental.pallas import tpu_sc as plsc

NC = 2
NS = 16
NW = NC * NS

V = 1000000
D = 64
B = 16384
H = 50

NCOLS = V // 128
VMAIN = NCOLS * 128
NPAIR = VMAIN // 2
COLS_PER_W = NCOLS // NW
COLS_EXTRA = NCOLS % NW
NUNITS = H * (B // 128)
UNITS_PER_W = NUNITS // NW

_mesh = lambda: plsc.VectorSubcoreMesh(core_axis_name="c", subcore_axis_name="s")


def _splat(val):
  return jnp.full((16,), val, jnp.int32)


def _build_convert():
  @functools.partial(
      pl.kernel,
      out_type=jax.ShapeDtypeStruct((NPAIR, 128), jnp.float32),
      mesh=_mesh(),
      compiler_params=pltpu.CompilerParams(use_tc_tiling_on_sc=True, needs_layout_passes=False),
      scratch_types=[
          pltpu.VMEM((2, 64, 128), jnp.float32),
          pltpu.VMEM((2, 64, 128), jnp.float32),
          [pltpu.SemaphoreType.DMA] * 2,
          [pltpu.SemaphoreType.DMA] * 2,
      ],
  )
  def conv_k(tT_hbm, t2_hbm, in_v, out_v, gsems, ssems):
    wid = lax.axis_index("s") * NC + lax.axis_index("c")

    def col_of(s):
      return wid + NW * s

    def issue_loads(s, b):
      j = col_of(s)
      for a in range(8):
        pltpu.async_copy(
            tT_hbm.at[pl.ds(8 * a, 8), pl.ds(128 * j, 128)],
            in_v.at[b, pl.ds(8 * a, 8)], gsems[b])

    def wait_loads(s, b):
      j = col_of(s)
      for a in range(8):
        pltpu.make_async_copy(
            tT_hbm.at[pl.ds(8 * a, 8), pl.ds(128 * j, 128)],
            in_v.at[b, pl.ds(8 * a, 8)], gsems[b]).wait()

    def store(s, b):
      j = col_of(s)
      return pltpu.make_async_copy(
          out_v.at[b], t2_hbm.at[pl.ds(64 * j, 64)], ssems[b])

    rows4 = [lax.iota(jnp.int32, 16) + 16 * qq for qq in range(4)]

    def transpose_col(b):
      def body(r, carry):
        for q in range(8):
          vec = plsc.load_gather(
              in_v.at[b], [rows4[q % 4], _splat(2 * r + q // 4)])
          out_v[b, r, pl.ds(16 * q, 16)] = vec
        return carry

      lax.fori_loop(0, 64, body, 0, unroll=2)

    issue_loads(0, 0)

    def group(g, carry):
      for b in range(2):
        s = 2 * g + b

        @pl.when(s + 1 < COLS_PER_W)
        def _():
          issue_loads(s + 1, 1 - b)

        wait_loads(s, b)

        @pl.when(s >= 2)
        def _():
          store(s - 2, b).wait()

        transpose_col(b)
        store(s, b).start()
      return carry

    lax.fori_loop(0, COLS_PER_W // 2, group, 0)
    store(COLS_PER_W - 2, 0).wait()
    store(COLS_PER_W - 1, 1).wait()

    @pl.when(wid < COLS_EXTRA)
    def _():
      j = NW * COLS_PER_W + wid
      for a in range(8):
        pltpu.async_copy(
            tT_hbm.at[pl.ds(8 * a, 8), pl.ds(128 * j, 128)],
            in_v.at[0, pl.ds(8 * a, 8)], gsems[0])
      for a in range(8):
        pltpu.make_async_copy(
            tT_hbm.at[pl.ds(8 * a, 8), pl.ds(128 * j, 128)],
            in_v.at[0, pl.ds(8 * a, 8)], gsems[0]).wait()
      transpose_col(0)
      pltpu.sync_copy(out_v.at[0], t2_hbm.at[pl.ds(64 * j, 64)])

  return conv_k


def _build_gather():
  @functools.partial(
      pl.kernel,
      out_type=jax.ShapeDtypeStruct((H, D, B), jnp.float32),
      mesh=_mesh(),
      compiler_params=pltpu.CompilerParams(use_tc_tiling_on_sc=True, needs_layout_passes=False),
      scratch_types=[
          pltpu.VMEM((2, 8, 128), jnp.int32),
          pltpu.VMEM((2, 128), jnp.int32),
          pltpu.VMEM((2, 160, 128), jnp.float32),
          pltpu.VMEM((2, 64, 128), jnp.float32),
          [pltpu.SemaphoreType.DMA] * 2,
          [pltpu.SemaphoreType.DMA] * 2,
          [pltpu.SemaphoreType.DMA] * 2,
      ],
  )
  def gath_k(xT_hbm, t2_hbm, tail_hbm, o3_hbm, xt_v, pidx_v, buf_v, tr_v,
             xsems, gsems, ssems):
    wid = lax.axis_index("s") * NC + lax.axis_index("c")
    pltpu.sync_copy(tail_hbm, buf_v.at[0, pl.ds(128, 32)])
    pltpu.sync_copy(tail_hbm, buf_v.at[1, pl.ds(128, 32)])

    def unit_of(s):
      u = wid + NW * s
      return u // 128, u % 128

    def issue_xload(s, b):
      h, j = unit_of(s)
      pltpu.async_copy(
          xT_hbm.at[pl.ds(8 * (h // 8), 8), pl.ds(128 * j, 128)],
          xt_v.at[b], xsems[b])

    def wait_xload(s, b):
      h, j = unit_of(s)
      pltpu.make_async_copy(
          xT_hbm.at[pl.ds(8 * (h // 8), 8), pl.ds(128 * j, 128)],
          xt_v.at[b], xsems[b]).wait()

    def issue_gather(s, b):
      h, _ = unit_of(s)
      hl = h % 8
      for q in range(8):
        idx = xt_v[b, hl, pl.ds(16 * q, 16)]
        pidx_v[b, pl.ds(16 * q, 16)] = (
            jnp.minimum(idx, _splat(VMAIN - 1)) >> 1)
      pltpu.async_copy(
          t2_hbm.at[pidx_v.at[b]], buf_v.at[b, pl.ds(0, 128)], gsems[b])

    def wait_gather(b):
      pltpu.make_async_copy(
          t2_hbm.at[pidx_v.at[b]], buf_v.at[b, pl.ds(0, 128)],
          gsems[b]).wait()

    def store(s, b):
      h, j = unit_of(s)
      return [
          pltpu.make_async_copy(
              tr_v.at[b, pl.ds(8 * i, 8)],
              o3_hbm.at[h, pl.ds(8 * i, 8), pl.ds(128 * j, 128)], ssems[b])
          for i in range(8)
      ]

    def transpose_unit(s, b):
      h, _ = unit_of(s)
      hl = h % 8
      rowvs = []
      colvs = []
      for q in range(8):
        idx = xt_v[b, hl, pl.ds(16 * q, 16)]
        is_tail = idx >= _splat(VMAIN)
        row = jnp.where(is_tail, (idx >> 1) - _splat(NPAIR - 128),
                        lax.iota(jnp.int32, 16) + 16 * q)
        col0 = (idx & _splat(1)) << 6
        rowvs.append(row)
        colvs.append(col0)

      def body(d, carry):
        for q in range(8):
          vec = plsc.load_gather(
              buf_v.at[b], [rowvs[q], colvs[q] + d])
          tr_v[b, d, pl.ds(16 * q, 16)] = vec
        return carry

      lax.fori_loop(0, 64, body, 0, unroll=2)

    issue_xload(0, 0)
    wait_xload(0, 0)
    issue_gather(0, 0)
    issue_xload(1, 1)

    def group(g, carry):
      for b in range(2):
        s = 2 * g + b

        @pl.when(s + 1 < UNITS_PER_W)
        def _():
          wait_xload(s + 1, 1 - b)
          issue_gather(s + 1, 1 - b)

        wait_gather(b)

        @pl.when(s >= 2)
        def _():
          for c in store(s - 2, b):
            c.wait()

        transpose_unit(s, b)
        for c in store(s, b):
          c.start()

        @pl.when(s + 2 < UNITS_PER_W)
        def _():
          issue_xload(s + 2, b)
      return carry

    lax.fori_loop(0, UNITS_PER_W // 2, group, 0)
    for c in store(UNITS_PER_W - 2, 0):
      c.wait()
    for c in store(UNITS_PER_W - 1, 1):
      c.wait()

  return gath_k


def kernel(x, table):
  tT = table.T
  xT = x.astype(jnp.int32).T
  tail = table[VMAIN:].reshape(32, 128)
  t2 = _build_convert()(tT)
  o3 = _build_gather()(xT, t2, tail)
  return o3.transpose(2, 0, 1)


# speedup vs baseline: 1.7927x; 1.7927x over previous
import functools

import jax
import jax.numpy as jnp
from jax import lax
from jax.experimental import pallas as pl
from jax.experimental.pallas import tpu as pltpu
from jax.experimental.pallas import tpu_sc as plsc

NC = 2
NS = 16
NW = NC * NS

V = 1000000
D = 64
B = 16384
H = 50

NCOLS = V // 128
VMAIN = NCOLS * 128
NPAIR = VMAIN // 2
COLS_PER_W = NCOLS // NW
COLS_EXTRA = NCOLS % NW
NUNITS = H * (B // 128)
UNITS_PER_W = NUNITS // NW

_mesh = lambda: plsc.VectorSubcoreMesh(core_axis_name="c", subcore_axis_name="s")


def _splat(val):
  return jnp.full((16,), val, jnp.int32)


def _build_convert():
  @functools.partial(
      pl.kernel,
      out_type=jax.ShapeDtypeStruct((NPAIR, 128), jnp.float32),
      mesh=_mesh(),
      compiler_params=pltpu.CompilerParams(use_tc_tiling_on_sc=True, needs_layout_passes=False),
      scratch_types=[
          pltpu.VMEM((2, 64, 128), jnp.float32),
          pltpu.VMEM((2, 64, 128), jnp.float32),
          [pltpu.SemaphoreType.DMA] * 2,
          [pltpu.SemaphoreType.DMA] * 2,
      ],
  )
  def conv_k(tT_hbm, t2_hbm, in_v, out_v, gsems, ssems):
    wid = lax.axis_index("s") * NC + lax.axis_index("c")

    def col_of(s):
      return wid + NW * s

    def issue_loads(s, b):
      j = col_of(s)
      for a in range(8):
        pltpu.async_copy(
            tT_hbm.at[pl.ds(8 * a, 8), pl.ds(128 * j, 128)],
            in_v.at[b, pl.ds(8 * a, 8)], gsems[b])

    def wait_loads(s, b):
      j = col_of(s)
      for a in range(8):
        pltpu.make_async_copy(
            tT_hbm.at[pl.ds(8 * a, 8), pl.ds(128 * j, 128)],
            in_v.at[b, pl.ds(8 * a, 8)], gsems[b]).wait()

    def store(s, b):
      j = col_of(s)
      return pltpu.make_async_copy(
          out_v.at[b], t2_hbm.at[pl.ds(64 * j, 64)], ssems[b])

    rows4 = [lax.iota(jnp.int32, 16) + 16 * qq for qq in range(4)]

    def transpose_col(b):
      @plsc.parallel_loop(0, 64, unroll=8)
      def _(r):
        for q in range(8):
          vec = plsc.load_gather(
              in_v.at[b], [rows4[q % 4], _splat(2 * r + q // 4)])
          out_v[b, r, pl.ds(16 * q, 16)] = vec

    issue_loads(0, 0)

    def group(g, carry):
      for b in range(2):
        s = 2 * g + b

        @pl.when(s + 1 < COLS_PER_W)
        def _():
          issue_loads(s + 1, 1 - b)

        wait_loads(s, b)

        @pl.when(s >= 2)
        def _():
          store(s - 2, b).wait()

        transpose_col(b)
        store(s, b).start()
      return carry

    lax.fori_loop(0, COLS_PER_W // 2, group, 0)
    store(COLS_PER_W - 2, 0).wait()
    store(COLS_PER_W - 1, 1).wait()

    @pl.when(wid < COLS_EXTRA)
    def _():
      j = NW * COLS_PER_W + wid
      for a in range(8):
        pltpu.async_copy(
            tT_hbm.at[pl.ds(8 * a, 8), pl.ds(128 * j, 128)],
            in_v.at[0, pl.ds(8 * a, 8)], gsems[0])
      for a in range(8):
        pltpu.make_async_copy(
            tT_hbm.at[pl.ds(8 * a, 8), pl.ds(128 * j, 128)],
            in_v.at[0, pl.ds(8 * a, 8)], gsems[0]).wait()
      transpose_col(0)
      pltpu.sync_copy(out_v.at[0], t2_hbm.at[pl.ds(64 * j, 64)])

  return conv_k


def _build_gather():
  @functools.partial(
      pl.kernel,
      out_type=jax.ShapeDtypeStruct((H, D, B), jnp.float32),
      mesh=_mesh(),
      compiler_params=pltpu.CompilerParams(use_tc_tiling_on_sc=True, needs_layout_passes=False),
      scratch_types=[
          pltpu.VMEM((2, 8, 128), jnp.int32),
          pltpu.VMEM((2, 128), jnp.int32),
          pltpu.VMEM((2, 160, 128), jnp.float32),
          pltpu.VMEM((2, 64, 128), jnp.float32),
          [pltpu.SemaphoreType.DMA] * 2,
          [pltpu.SemaphoreType.DMA] * 2,
          [pltpu.SemaphoreType.DMA] * 2,
      ],
  )
  def gath_k(xT_hbm, t2_hbm, tail_hbm, o3_hbm, xt_v, pidx_v, buf_v, tr_v,
             xsems, gsems, ssems):
    wid = lax.axis_index("s") * NC + lax.axis_index("c")
    pltpu.sync_copy(tail_hbm, buf_v.at[0, pl.ds(128, 32)])
    pltpu.sync_copy(tail_hbm, buf_v.at[1, pl.ds(128, 32)])

    def unit_of(s):
      u = wid + NW * s
      return u // 128, u % 128

    def issue_xload(s, b):
      h, j = unit_of(s)
      pltpu.async_copy(
          xT_hbm.at[pl.ds(8 * (h // 8), 8), pl.ds(128 * j, 128)],
          xt_v.at[b], xsems[b])

    def wait_xload(s, b):
      h, j = unit_of(s)
      pltpu.make_async_copy(
          xT_hbm.at[pl.ds(8 * (h // 8), 8), pl.ds(128 * j, 128)],
          xt_v.at[b], xsems[b]).wait()

    def issue_gather(s, b):
      h, _ = unit_of(s)
      hl = h % 8
      for q in range(8):
        idx = xt_v[b, hl, pl.ds(16 * q, 16)]
        pidx_v[b, pl.ds(16 * q, 16)] = (
            jnp.minimum(idx, _splat(VMAIN - 1)) >> 1)
      pltpu.async_copy(
          t2_hbm.at[pidx_v.at[b]], buf_v.at[b, pl.ds(0, 128)], gsems[b])

    def wait_gather(b):
      pltpu.make_async_copy(
          t2_hbm.at[pidx_v.at[b]], buf_v.at[b, pl.ds(0, 128)],
          gsems[b]).wait()

    def store(s, b):
      h, j = unit_of(s)
      return [
          pltpu.make_async_copy(
              tr_v.at[b, pl.ds(8 * i, 8)],
              o3_hbm.at[h, pl.ds(8 * i, 8), pl.ds(128 * j, 128)], ssems[b])
          for i in range(8)
      ]

    def transpose_unit(s, b):
      h, _ = unit_of(s)
      hl = h % 8
      rowvs = []
      colvs = []
      for q in range(8):
        idx = xt_v[b, hl, pl.ds(16 * q, 16)]
        is_tail = idx >= _splat(VMAIN)
        row = jnp.where(is_tail, (idx >> 1) - _splat(NPAIR - 128),
                        lax.iota(jnp.int32, 16) + 16 * q)
        col0 = (idx & _splat(1)) << 6
        rowvs.append(row)
        colvs.append(col0)

      @plsc.parallel_loop(0, 64, unroll=8)
      def _(d):
        for q in range(8):
          vec = plsc.load_gather(
              buf_v.at[b], [rowvs[q], colvs[q] + d])
          tr_v[b, d, pl.ds(16 * q, 16)] = vec

    issue_xload(0, 0)
    wait_xload(0, 0)
    issue_gather(0, 0)
    issue_xload(1, 1)

    def group(g, carry):
      for b in range(2):
        s = 2 * g + b

        @pl.when(s + 1 < UNITS_PER_W)
        def _():
          wait_xload(s + 1, 1 - b)
          issue_gather(s + 1, 1 - b)

        wait_gather(b)

        @pl.when(s >= 2)
        def _():
          for c in store(s - 2, b):
            c.wait()

        transpose_unit(s, b)
        for c in store(s, b):
          c.start()

        @pl.when(s + 2 < UNITS_PER_W)
        def _():
          issue_xload(s + 2, b)
      return carry

    lax.fori_loop(0, UNITS_PER_W // 2, group, 0)
    for c in store(UNITS_PER_W - 2, 0):
      c.wait()
    for c in store(UNITS_PER_W - 1, 1):
      c.wait()

  return gath_k


def kernel(x, table):
  tT = table.T
  xT = x.astype(jnp.int32).T
  tail = table[VMAIN:].reshape(32, 128)
  t2 = _build_convert()(tT)
  o3 = _build_gather()(xT, t2, tail)
  return o3.transpose(2, 0, 1)
